# baseline (speedup 1.0000x reference)
import functools

import jax
import jax.numpy as jnp
from jax import lax
from jax.experimental import pallas as pl
from jax.experimental.pallas import tpu as pltpu
from jax.experimental.pallas import tpu_sc as plsc

_NC, _NS = 2, 16
_NW = _NC * _NS
_CHUNK = 128


def _sc_gather4(user_ids, item_ids, gu_t, gi_t, mu_t, mi_t):
    B = user_ids.shape[0]
    D = gu_t.shape[1]
    b_per_w = B // _NW
    n_chunks = b_per_w // _CHUNK
    uid3 = user_ids.reshape(_NW, n_chunks, _CHUNK)
    iid3 = item_ids.reshape(_NW, n_chunks, _CHUNK)
    mesh = plsc.VectorSubcoreMesh(core_axis_name="c", subcore_axis_name="s")
    row_t = jax.ShapeDtypeStruct((B, D), jnp.float32)

    @functools.partial(
        pl.kernel,
        mesh=mesh,
        compiler_params=pltpu.CompilerParams(use_tc_tiling_on_sc=False),
        out_type=[row_t, row_t, row_t, row_t],
        scratch_types=[
            pltpu.VMEM((n_chunks, _CHUNK), jnp.int32),
            pltpu.VMEM((n_chunks, _CHUNK), jnp.int32),
            pltpu.VMEM((b_per_w, D), jnp.float32),
            pltpu.VMEM((b_per_w, D), jnp.float32),
            pltpu.VMEM((b_per_w, D), jnp.float32),
            pltpu.VMEM((b_per_w, D), jnp.float32),
            pltpu.SemaphoreType.DMA,
        ],
    )
    def k(uid_hbm, iid_hbm, gu_hbm, gi_hbm, mu_hbm, mi_hbm,
          o0, o1, o2, o3, idx_u, idx_i, r0, r1, r2, r3, sem):
        wid = lax.axis_index("s") * _NC + lax.axis_index("c")
        pltpu.sync_copy(uid_hbm.at[wid], idx_u)
        pltpu.sync_copy(iid_hbm.at[wid], idx_i)
        copies = []
        for c in range(n_chunks):
            dst = pl.ds(c * _CHUNK, _CHUNK)
            copies.append(pltpu.async_copy(gu_hbm.at[idx_u.at[c]], r0.at[dst], sem))
            copies.append(pltpu.async_copy(gi_hbm.at[idx_i.at[c]], r1.at[dst], sem))
            copies.append(pltpu.async_copy(mu_hbm.at[idx_u.at[c]], r2.at[dst], sem))
            copies.append(pltpu.async_copy(mi_hbm.at[idx_i.at[c]], r3.at[dst], sem))
        for cp in copies:
            cp.wait()
        out = pl.ds(wid * b_per_w, b_per_w)
        pltpu.sync_copy(r0, o0.at[out])
        pltpu.sync_copy(r1, o1.at[out])
        pltpu.sync_copy(r2, o2.at[out])
        pltpu.sync_copy(r3, o3.at[out])

    return k(uid3, iid3, gu_t, gi_t, mu_t, mi_t)


def _tc_body(gu, gi, mu, mi, w0, b0r, w1, b1r, w2, b2r, fw, fb, out):
    d = mu.shape[1]
    w0v = w0[...]
    h = jnp.dot(mu[...], w0v[:d, :], preferred_element_type=jnp.float32)
    h = h + jnp.dot(mi[...], w0v[d:, :], preferred_element_type=jnp.float32)
    h = jnp.maximum(h + b0r[...], 0.0)
    h = jnp.dot(h, w1[...], preferred_element_type=jnp.float32) + b1r[...]
    h = jnp.maximum(h, 0.0)
    h = jnp.dot(h, w2[...], preferred_element_type=jnp.float32) + b2r[...]
    h = jnp.maximum(h, 0.0)
    fwv = fw[...]
    r = jnp.dot(gu[...] * gi[...], fwv[:d, :], preferred_element_type=jnp.float32)
    r = r + jnp.dot(h, fwv[d:, :], preferred_element_type=jnp.float32)
    out[...] = r + fb[...]


def _tc_mlp(gmf_u, gmf_i, mlp_u, mlp_i, W0, b0, W1, b1, W2, b2, fc_w, fc_b):
    B, D = gmf_u.shape
    blk = 2048

    def full(a):
        return pl.BlockSpec(a.shape, lambda i: (0,) * a.ndim)

    row_spec = pl.BlockSpec((blk, D), lambda i: (i, 0))
    out = pl.pallas_call(
        _tc_body,
        grid=(B // blk,),
        in_specs=[row_spec, row_spec, row_spec, row_spec,
                  full(W0), full(b0), full(W1), full(b1), full(W2), full(b2),
                  full(fc_w), full(fc_b)],
        out_specs=pl.BlockSpec((blk, 1), lambda i: (i, 0)),
        out_shape=jax.ShapeDtypeStruct((B, 1), jnp.float32),
    )(gmf_u, gmf_i, mlp_u, mlp_i, W0, b0, W1, b1, W2, b2, fc_w, fc_b)
    return jnp.squeeze(out, axis=-1)


def kernel(user_ids, item_ids, gmf_user_emb, gmf_item_emb, mlp_user_emb,
           mlp_item_emb, W0, b0, W1, b1, W2, b2, fc_w, fc_b):
    gu, gi, mu, mi = _sc_gather4(user_ids, item_ids, gmf_user_emb,
                                 gmf_item_emb, mlp_user_emb, mlp_item_emb)
    return _tc_mlp(gu, gi, mu, mi, W0, b0, W1, b1, W2, b2, fc_w, fc_b)


# speedup vs baseline: 2.8744x; 2.8744x over previous
import functools

import jax
import jax.numpy as jnp
from jax import lax
from jax.experimental import pallas as pl
from jax.experimental.pallas import tpu as pltpu
from jax.experimental.pallas import tpu_sc as plsc

_NC, _NS = 2, 16
_NW = _NC * _NS
_CHUNK = 64
_RING = 16


def _sc_gather4(user_ids, item_ids, gu_t, mu_t, gi_g, mi_g):
    B = user_ids.shape[0]
    b_per_w = B // _NW
    n_chunks = b_per_w // _CHUNK
    uid2 = user_ids.reshape(_NW, b_per_w)
    iid3 = item_ids.reshape(_NW, n_chunks, _CHUNK)
    mesh = plsc.VectorSubcoreMesh(core_axis_name="c", subcore_axis_name="s")
    row_t = jax.ShapeDtypeStruct((B, 16), jnp.float32)
    gran_t = jax.ShapeDtypeStruct((B, 128), jnp.float32)

    @functools.partial(
        pl.kernel,
        mesh=mesh,
        compiler_params=pltpu.CompilerParams(needs_layout_passes=False),
        out_type=[row_t, row_t, gran_t, gran_t],
        scratch_types=[
            pltpu.VMEM((b_per_w,), jnp.int32),
            pltpu.VMEM((n_chunks, _CHUNK), jnp.int32),
            pltpu.VMEM((b_per_w, 16), jnp.float32),
            [pltpu.VMEM((16, 128), jnp.float32) for _ in range(_RING)],
            pltpu.VMEM((_CHUNK, 128), jnp.float32),
            pltpu.VMEM((_CHUNK, 128), jnp.float32),
            [pltpu.SemaphoreType.DMA for _ in range(_RING)],
            pltpu.SemaphoreType.DMA,
        ],
    )
    def k(uid_hbm, iid_hbm, gu_hbm, mu_hbm, gi_hbm, mi_hbm,
          o_gu, o_mu, o_gi, o_mi, u_sm, idx_i, stage, ring, ri0, ri1,
          ring_sems, isem):
        wid = lax.axis_index("s") * _NC + lax.axis_index("c")
        pltpu.sync_copy(uid_hbm.at[wid], u_sm)
        pltpu.sync_copy(iid_hbm.at[wid], idx_i)

        @pl.loop(0, n_chunks)
        def _shift(c):
            row = idx_i.at[c]

            @pl.loop(0, _CHUNK // 16)
            def _s16(l):
                s = pl.ds(l * 16, 16)
                row[s] = row[s] >> 3

        @pl.loop(0, n_chunks)
        def _chunk(c):
            pltpu.async_copy(gi_hbm.at[idx_i.at[c]], ri0, isem)
            pltpu.async_copy(mi_hbm.at[idx_i.at[c]], ri1, isem)
            pltpu.make_async_copy(gi_hbm.at[idx_i.at[c]], ri0, isem).wait()
            pltpu.make_async_copy(mi_hbm.at[idx_i.at[c]], ri1, isem).wait()
            out = pl.ds(wid * b_per_w + c * _CHUNK, _CHUNK)
            pltpu.sync_copy(ri0, o_gi.at[out])
            pltpu.sync_copy(ri1, o_mi.at[out])

        lanes = lax.iota(jnp.int32, 16)
        n_grp = b_per_w // _RING

        def user_table(tab_hbm, o_ref):
            def fire(b, uid_scalar):
                col = pl.multiple_of((uid_scalar >> 7) * 128, 128)
                pltpu.async_copy(tab_hbm.at[:, pl.ds(col, 128)], ring[b],
                                 ring_sems[b])

            def wait(b):
                pltpu.make_async_copy(tab_hbm.at[:, pl.ds(0, 128)], ring[b],
                                      ring_sems[b]).wait()

            ids0 = u_sm[pl.ds(0, _RING)]
            for b in range(_RING):
                fire(b, ids0[b])

            @pl.loop(0, n_grp)
            def _grp(g):
                base = g * _RING
                ids_cur = u_sm[pl.ds(base, _RING)]
                nxt = ((g + 1) % n_grp) * _RING
                ids_nxt = u_sm[pl.ds(nxt, _RING)]
                for b in range(_RING):
                    wait(b)
                    lane = jnp.full((16,), ids_cur[b] & 127, jnp.int32)
                    vals = plsc.load_gather(ring[b], [lanes, lane])
                    stage.at[base + b][...] = vals
                    fire(b, ids_nxt[b])

            for b in range(_RING):
                wait(b)
            pltpu.sync_copy(stage, o_ref.at[pl.ds(wid * b_per_w, b_per_w)])

        user_table(gu_hbm, o_gu)
        user_table(mu_hbm, o_mu)

    return k(uid2, iid3, gu_t, mu_t, gi_g, mi_g)


def _tc_body(iid, gu, mu, gi_g, mi_g, w0u, w0i, b0r, w1, b1r, w2, b2r,
             fwg, fwh, fb, out):
    ji = iid[...] & 7
    gi = None
    mi = None
    for j in range(8):
        s = slice(16 * j, 16 * j + 16)
        m = ji[:, None] == j
        pg = jnp.where(m, gi_g[...][:, s], 0.0)
        pm = jnp.where(m, mi_g[...][:, s], 0.0)
        gi = pg if gi is None else gi + pg
        mi = pm if mi is None else mi + pm
    gmf = gu[...] * gi
    h = jnp.dot(mu[...], w0u[...], preferred_element_type=jnp.float32)
    h = h + jnp.dot(mi, w0i[...], preferred_element_type=jnp.float32)
    h = jnp.maximum(h + b0r[...], 0.0)
    h = jnp.maximum(jnp.dot(h, w1[...], preferred_element_type=jnp.float32)
                    + b1r[...], 0.0)
    h = jnp.maximum(jnp.dot(h, w2[...], preferred_element_type=jnp.float32)
                    + b2r[...], 0.0)
    r = jnp.dot(gmf, fwg[...], preferred_element_type=jnp.float32)
    r = r + jnp.dot(h, fwh[...], preferred_element_type=jnp.float32)
    out[...] = r + fb[...]


def _tc_mlp(item_ids, gu, mu, gi_g, mi_g, W0, b0, W1, b1, W2, b2, fc_w,
            fc_b):
    B = gu.shape[0]
    blk = 2048
    d = 16
    w0u, w0i = W0[:d], W0[d:]
    fwg, fwh = fc_w[:d], fc_w[d:]
    b0r = b0.reshape(1, -1)
    b1r = b1.reshape(1, -1)
    b2r = b2.reshape(1, -1)
    fbr = fc_b.reshape(1, 1)

    def full(a):
        return pl.BlockSpec(a.shape, lambda i: (0,) * a.ndim)

    id_spec = pl.BlockSpec((blk,), lambda i: (i,))
    row_spec = pl.BlockSpec((blk, d), lambda i: (i, 0))
    g_spec = pl.BlockSpec((blk, 128), lambda i: (i, 0))
    out = pl.pallas_call(
        _tc_body,
        grid=(B // blk,),
        in_specs=[id_spec, row_spec, row_spec, g_spec, g_spec,
                  full(w0u), full(w0i), full(b0r), full(W1), full(b1r),
                  full(W2), full(b2r), full(fwg), full(fwh), full(fbr)],
        out_specs=pl.BlockSpec((blk, 1), lambda i: (i, 0)),
        out_shape=jax.ShapeDtypeStruct((B, 1), jnp.float32),
    )(item_ids, gu, mu, gi_g, mi_g, w0u, w0i, b0r, W1, b1r, W2, b2r, fwg,
      fwh, fbr)
    return out.reshape(B)


def kernel(user_ids, item_ids, gmf_user_emb, gmf_item_emb, mlp_user_emb,
           mlp_item_emb, W0, b0, W1, b1, W2, b2, fc_w, fc_b):
    gi_g = gmf_item_emb.reshape(-1, 128)
    mi_g = mlp_item_emb.reshape(-1, 128)
    gu, mu, gi_gr, mi_gr = _sc_gather4(
        user_ids, item_ids, gmf_user_emb.T, mlp_user_emb.T, gi_g, mi_g)
    return _tc_mlp(item_ids, gu, mu, gi_gr, mi_gr, W0, b0, W1, b1, W2, b2,
                   fc_w, fc_b)
